# Initial kernel scaffold; baseline (speedup 1.0000x reference)
import jax
import jax.numpy as jnp
from jax.experimental import pallas as pl


def kernel(h, edge_index, W, a):
    raise NotImplementedError("write your pallas kernel here")


# speedup vs baseline: 35.3930x; 35.3930x over previous
import functools

import jax
import jax.numpy as jnp
from jax import lax
from jax.experimental import pallas as pl
from jax.experimental.pallas import tpu as pltpu
from jax.experimental.pallas import tpu_sc as plsc

N = 10000
E = 320000
F = 128
NC = 2
NS = 16
NW = NC * NS
C = 80
NCHUNK = E // C
NMY = NCHUNK // NW
NB = 3
RSPAN = 624
RBLK = 48
RTAIL = N - NS * RSPAN
NEG_SLOPE = 0.2


def _proj_body(h_ref, w_ref, a2_ref, p_ref, s_ref, m_ref):
    p = jnp.dot(h_ref[...], w_ref[0], preferred_element_type=jnp.float32)
    p_ref[...] = p
    s = jnp.dot(p, a2_ref[...], preferred_element_type=jnp.float32)
    s_ref[...] = s
    m = jnp.max(s[:, 0]) + jnp.max(s[:, 1])
    m = jnp.where(m >= 0, m, NEG_SLOPE * m)
    m_ref[...] = jnp.full((1, 128), m, dtype=jnp.float32)


def _project(h, w, a2):
    return pl.pallas_call(
        _proj_body,
        out_shape=[
            jax.ShapeDtypeStruct((N, F), jnp.float32),
            jax.ShapeDtypeStruct((N, F), jnp.float32),
            jax.ShapeDtypeStruct((1, 128), jnp.float32),
        ],
    )(h, w, a2)


def _edges_body(ei_hbm, p_hbm, s1_hbm, s2_hbm, m_hbm,
                acc_out, den_out,
                den_v, m_v, src_v, dst_v, sdst_v, w_v, g1_v, g2_v, rows_v,
                si, sg, sgs, ss,
                acc_sh, den_sh):
    cid = lax.axis_index("c")
    sid = lax.axis_index("s")
    wid = sid * NC + cid

    pltpu.sync_copy(m_hbm, m_v)

    zero16 = jnp.zeros((16,), jnp.float32)

    def _zrow(i, _):
        for q in range(F // 16):
            rows_v[0, i, pl.ds(q * 16, 16)] = zero16
        return 0
    lax.fori_loop(0, RBLK, _zrow, 0)

    row0 = sid * RSPAN
    for blk in range(RSPAN // RBLK):
        pltpu.sync_copy(rows_v.at[0, pl.ds(0, RBLK)],
                        acc_sh.at[pl.ds(row0 + blk * RBLK, RBLK)])

    @pl.when(sid == NS - 1)
    def _():
        pltpu.sync_copy(rows_v.at[0, pl.ds(0, RTAIL)],
                        acc_sh.at[pl.ds(NS * RSPAN, RTAIL)])

    def _zden(i, _):
        den_v[pl.ds(i * 16, 16)] = zero16
        return 0
    lax.fori_loop(0, N // 16, _zden, 0)

    @pl.when(sid == 0)
    def _():
        pltpu.sync_copy(den_v, den_sh)

    plsc.subcore_barrier()

    mvec = m_v[...]

    def _issue_idx(b, j):
        off = (wid + j * NW) * C
        pltpu.async_copy(ei_hbm.at[pl.ds(off, C)], src_v.at[b], si[b])
        pltpu.async_copy(ei_hbm.at[pl.ds(E + off, C)], dst_v.at[b], si[b])

    def _wait_idx(b):
        pltpu.make_async_copy(ei_hbm.at[pl.ds(0, C)], src_v.at[b],
                              si[b]).wait()
        pltpu.make_async_copy(ei_hbm.at[pl.ds(0, C)], dst_v.at[b],
                              si[b]).wait()

    def _wait_scatter(b):
        pltpu.make_async_copy(rows_v.at[b], acc_sh.at[sdst_v.at[b]],
                              ss[b]).wait()
        pltpu.make_async_copy(w_v.at[b], den_sh.at[sdst_v.at[b]],
                              ss[b]).wait()

    def _issue_gather(b):
        pltpu.async_copy(p_hbm.at[src_v.at[b]], rows_v.at[b], sg[b])
        pltpu.async_copy(s1_hbm.at[src_v.at[b]], g1_v.at[b], sgs[b])
        pltpu.async_copy(s2_hbm.at[dst_v.at[b]], g2_v.at[b], sgs[b])

    def _wait_gather(b):
        pltpu.make_async_copy(p_hbm.at[src_v.at[b]], rows_v.at[b],
                              sg[b]).wait()
        pltpu.make_async_copy(s1_hbm.at[src_v.at[b]], g1_v.at[b],
                              sgs[b]).wait()
        pltpu.make_async_copy(s2_hbm.at[dst_v.at[b]], g2_v.at[b],
                              sgs[b]).wait()

    def _do_chunk(j, b, has_next, has_next2):
        b1 = (b + 1) % NB
        b2 = (b + 2) % NB
        _wait_gather(b)

        def _wgrp(g, _):
            t = g1_v[b, pl.ds(g * 16, 16)] + g2_v[b, pl.ds(g * 16, 16)]
            t = jnp.where(t >= 0, t, NEG_SLOPE * t)
            w_v[b, pl.ds(g * 16, 16)] = jnp.exp(t - mvec)
            sdst_v[b, pl.ds(g * 16, 16)] = dst_v[b, pl.ds(g * 16, 16)]
            return 0
        lax.fori_loop(0, C // 16, _wgrp, 0)

        if isinstance(j, int):
            if j >= 2:
                _wait_scatter(b1)
        else:
            @pl.when(j >= 2)
            def _():
                _wait_scatter(b1)

        if has_next:
            _wait_idx(b1)
            _issue_gather(b1)

        if has_next2:
            _issue_idx(b2, j + 2)

        def _scale(g, _):
            w16 = w_v[b, pl.ds(g * 16, 16)]
            for l in range(16):
                ws = w16[l]
                e = g * 16 + l
                for q in range(F // 16):
                    rows_v[b, e, pl.ds(q * 16, 16)] = (
                        rows_v[b, e, pl.ds(q * 16, 16)] * ws)
            return 0
        lax.fori_loop(0, C // 16, _scale, 0)

        pltpu.async_copy(rows_v.at[b], acc_sh.at[sdst_v.at[b]], ss[b],
                         add=True)
        pltpu.async_copy(w_v.at[b], den_sh.at[sdst_v.at[b]], ss[b],
                         add=True)

    _issue_idx(0, 0)
    _wait_idx(0)
    _issue_gather(0)
    _issue_idx(1, 1)

    def _triple(t, _):
        _do_chunk(3 * t, 0, True, True)
        _do_chunk(3 * t + 1, 1, True, True)
        _do_chunk(3 * t + 2, 2, True, True)
        return 0
    lax.fori_loop(0, NMY // NB, _triple, 0)

    _do_chunk(NMY - 2, 0, True, False)
    _do_chunk(NMY - 1, 1, False, False)

    _wait_scatter(0)
    _wait_scatter(1)

    plsc.subcore_barrier()

    for blk in range(RSPAN // RBLK):
        r = row0 + blk * RBLK
        pltpu.sync_copy(acc_sh.at[pl.ds(r, RBLK)],
                        rows_v.at[0, pl.ds(0, RBLK)])
        pltpu.sync_copy(rows_v.at[0, pl.ds(0, RBLK)],
                        acc_out.at[cid, pl.ds(r, RBLK)])

    @pl.when(sid == NS - 1)
    def _():
        r = NS * RSPAN
        pltpu.sync_copy(acc_sh.at[pl.ds(r, RTAIL)],
                        rows_v.at[0, pl.ds(0, RTAIL)])
        pltpu.sync_copy(rows_v.at[0, pl.ds(0, RTAIL)],
                        acc_out.at[cid, pl.ds(r, RTAIL)])

    @pl.when(sid == 0)
    def _():
        pltpu.sync_copy(den_sh, den_v)
        pltpu.sync_copy(den_v, den_out.at[pl.ds(cid * N, N)])


@functools.partial(
    pl.kernel,
    out_type=[
        jax.ShapeDtypeStruct((NC, N, F), jnp.float32),
        jax.ShapeDtypeStruct((NC * N,), jnp.float32),
    ],
    mesh=plsc.VectorSubcoreMesh(core_axis_name="c", subcore_axis_name="s"),
    scratch_types=[
        pltpu.VMEM((N,), jnp.float32),
        pltpu.VMEM((16,), jnp.float32),
        pltpu.VMEM((NB, C), jnp.int32),
        pltpu.VMEM((NB, C), jnp.int32),
        pltpu.VMEM((NB, C), jnp.int32),
        pltpu.VMEM((NB, C), jnp.float32),
        pltpu.VMEM((NB, C), jnp.float32),
        pltpu.VMEM((NB, C), jnp.float32),
        pltpu.VMEM((NB, C, F), jnp.float32),
        pltpu.SemaphoreType.DMA,
        pltpu.SemaphoreType.DMA,
        pltpu.SemaphoreType.DMA,
        pltpu.SemaphoreType.DMA,
        pltpu.SemaphoreType.DMA,
        pltpu.SemaphoreType.DMA,
        pltpu.SemaphoreType.DMA,
        pltpu.SemaphoreType.DMA,
        pltpu.SemaphoreType.DMA,
        pltpu.SemaphoreType.DMA,
        pltpu.SemaphoreType.DMA,
        pltpu.SemaphoreType.DMA,
        pltpu.VMEM_SHARED((N, F), jnp.float32),
        pltpu.VMEM_SHARED((N,), jnp.float32),
    ],
)
def _edges(ei_hbm, p_hbm, s1_hbm, s2_hbm, m_hbm, acc_out, den_out,
           den_v, m_v, src_v, dst_v, sdst_v, w_v, g1_v, g2_v, rows_v,
           si0, si1, si2, sg0, sg1, sg2, sgs0, sgs1, sgs2, ss0, ss1, ss2,
           acc_sh, den_sh):
    _edges_body(ei_hbm, p_hbm, s1_hbm, s2_hbm, m_hbm,
                acc_out, den_out,
                den_v, m_v, src_v, dst_v, sdst_v, w_v, g1_v, g2_v, rows_v,
                (si0, si1, si2), (sg0, sg1, sg2), (sgs0, sgs1, sgs2),
                (ss0, ss1, ss2),
                acc_sh, den_sh)


def _finish_body(acc_ref, den_ref, o_ref):
    den = den_ref[0] + den_ref[1]
    acc = acc_ref[0] + acc_ref[1]
    r = acc / jnp.maximum(den, 1e-16)[:, None]
    o_ref[...] = jnp.where(r > 0, r, jnp.exp(jnp.minimum(r, 0.0)) - 1.0)


def _finish(acc, den):
    return pl.pallas_call(
        _finish_body,
        out_shape=jax.ShapeDtypeStruct((N, F), jnp.float32),
    )(acc, den)


def kernel(h, edge_index, W, a):
    a_pad = jnp.zeros((F, F), jnp.float32)
    a_pad = a_pad.at[:, 0].set(a[0, :F]).at[:, 1].set(a[0, F:])
    p, s_pad, m128 = _project(h, W, a_pad)
    s1 = s_pad[:, 0]
    s2 = s_pad[:, 1]
    m16 = m128[0, :16]
    acc, den = _edges(edge_index.reshape(2 * E), p, s1, s2, m16)
    return _finish(acc, den.reshape(NC, N))


# speedup vs baseline: 38.6221x; 1.0912x over previous
import functools

import jax
import jax.numpy as jnp
from jax import lax
from jax.experimental import pallas as pl
from jax.experimental.pallas import tpu as pltpu
from jax.experimental.pallas import tpu_sc as plsc

N = 10000
E = 320000
F = 128
NC = 2
NS = 16
NW = NC * NS
C = 128
NCHUNK = E // C
RSPAN = 624
RBLK = 104
RTAIL = N - NS * RSPAN
NEG_SLOPE = 0.2


def _proj_body(h_ref, w_ref, a2_ref, p_ref, s_ref, m_ref):
    p = jnp.dot(h_ref[...], w_ref[0], preferred_element_type=jnp.float32)
    p_ref[...] = p
    s = jnp.dot(p, a2_ref[...], preferred_element_type=jnp.float32)
    s_ref[...] = s
    m = jnp.max(s[:, 0]) + jnp.max(s[:, 1])
    m = jnp.where(m >= 0, m, NEG_SLOPE * m)
    m_ref[...] = jnp.full((1, 128), m, dtype=jnp.float32)


def _project(h, w, a2):
    return pl.pallas_call(
        _proj_body,
        out_shape=[
            jax.ShapeDtypeStruct((N, F), jnp.float32),
            jax.ShapeDtypeStruct((N, F), jnp.float32),
            jax.ShapeDtypeStruct((1, 128), jnp.float32),
        ],
    )(h, w, a2)


def _edges_body(ei_hbm, p_hbm, s1_hbm, s2_hbm, m_hbm,
                acc_out, den_out,
                den_v, m_v, sd_v, sdst_v, w_v, g1_v, g2_v, rows_v,
                si0, si1, sg0, sg1, sgs0, sgs1, ss0, ss1,
                acc_sh, den_sh):
    cid = lax.axis_index("c")
    sid = lax.axis_index("s")
    wid = sid * NC + cid
    si = (si0, si1)
    sg = (sg0, sg1)
    sgs = (sgs0, sgs1)
    ss = (ss0, ss1)

    pltpu.sync_copy(m_hbm, m_v)

    zero16 = jnp.zeros((16,), jnp.float32)

    def _zrow(i, _):
        for q in range(F // 16):
            rows_v[0, i, pl.ds(q * 16, 16)] = zero16
        return 0
    lax.fori_loop(0, RBLK, _zrow, 0)

    row0 = sid * RSPAN
    for blk in range(RSPAN // RBLK):
        pltpu.sync_copy(rows_v.at[0, pl.ds(0, RBLK)],
                        acc_sh.at[pl.ds(row0 + blk * RBLK, RBLK)])

    @pl.when(sid == NS - 1)
    def _():
        pltpu.sync_copy(rows_v.at[0, pl.ds(0, RTAIL)],
                        acc_sh.at[pl.ds(NS * RSPAN, RTAIL)])

    def _zden(i, _):
        den_v[pl.ds(i * 16, 16)] = zero16
        return 0
    lax.fori_loop(0, N // 16, _zden, 0)

    @pl.when(sid == 0)
    def _():
        pltpu.sync_copy(den_v, den_sh)

    plsc.subcore_barrier()

    mvec = m_v[...]
    n_my = (NCHUNK - wid + NW - 1) // NW

    def _issue_idx(b, j):
        off = (wid + j * NW) * 2 * C
        pltpu.async_copy(ei_hbm.at[pl.ds(off, 2 * C)], sd_v.at[b], si[b])

    def _wait_idx(b):
        pltpu.make_async_copy(ei_hbm.at[pl.ds(0, 2 * C)], sd_v.at[b],
                              si[b]).wait()

    def _wait_scatter(b):
        pltpu.make_async_copy(rows_v.at[b], acc_sh.at[sdst_v.at[b]],
                              ss[b]).wait()
        pltpu.make_async_copy(w_v.at[b], den_sh.at[sdst_v.at[b]],
                              ss[b]).wait()

    def _issue_gather(b):
        pltpu.async_copy(p_hbm.at[sd_v.at[b, pl.ds(0, C)]], rows_v.at[b],
                         sg[b])
        pltpu.async_copy(s1_hbm.at[sd_v.at[b, pl.ds(0, C)]], g1_v.at[b],
                         sgs[b])
        pltpu.async_copy(s2_hbm.at[sd_v.at[b, pl.ds(C, C)]], g2_v.at[b],
                         sgs[b])

    def _wait_gather(b):
        pltpu.make_async_copy(p_hbm.at[sd_v.at[b, pl.ds(0, C)]],
                              rows_v.at[b], sg[b]).wait()
        pltpu.make_async_copy(s1_hbm.at[sd_v.at[b, pl.ds(0, C)]],
                              g1_v.at[b], sgs[b]).wait()
        pltpu.make_async_copy(s2_hbm.at[sd_v.at[b, pl.ds(C, C)]],
                              g2_v.at[b], sgs[b]).wait()

    def _do_chunk(j, b):
        bn = 1 - b
        _wait_gather(b)

        def _wgrp(g, _):
            t = g1_v[b, pl.ds(g * 16, 16)] + g2_v[b, pl.ds(g * 16, 16)]
            t = jnp.where(t >= 0, t, NEG_SLOPE * t)
            w_v[b, pl.ds(g * 16, 16)] = jnp.exp(t - mvec)
            sdst_v[b, pl.ds(g * 16, 16)] = sd_v[b, pl.ds(C + g * 16, 16)]
            return 0
        lax.fori_loop(0, C // 16, _wgrp, 0)

        @pl.when(jnp.logical_and(j >= 1, j + 1 < n_my))
        def _():
            _wait_scatter(bn)

        @pl.when(j + 1 < n_my)
        def _():
            _wait_idx(bn)
            _issue_gather(bn)

        @pl.when(j + 2 < n_my)
        def _():
            _issue_idx(b, j + 2)

        def _scale(g, _):
            w16 = w_v[b, pl.ds(g * 16, 16)]
            for l in range(16):
                ws = w16[l]
                e = g * 16 + l
                for q in range(F // 16):
                    rows_v[b, e, pl.ds(q * 16, 16)] = (
                        rows_v[b, e, pl.ds(q * 16, 16)] * ws)
            return 0
        lax.fori_loop(0, C // 16, _scale, 0)

        pltpu.async_copy(rows_v.at[b], acc_sh.at[sdst_v.at[b]], ss[b],
                         add=True)
        pltpu.async_copy(w_v.at[b], den_sh.at[sdst_v.at[b]], ss[b],
                         add=True)

    _issue_idx(0, 0)
    _wait_idx(0)
    _issue_gather(0)
    @pl.when(n_my >= 2)
    def _():
        _issue_idx(1, 1)

    def _pair(p2, _):
        _do_chunk(2 * p2, 0)
        _do_chunk(2 * p2 + 1, 1)
        return 0
    lax.fori_loop(0, NCHUNK // NW // 2, _pair, 0)

    @pl.when(n_my % 2 == 1)
    def _():
        _do_chunk(n_my - 1, 0)

    _wait_scatter(0)
    _wait_scatter(1)

    plsc.subcore_barrier()

    for blk in range(RSPAN // RBLK):
        r = row0 + blk * RBLK
        pltpu.sync_copy(acc_sh.at[pl.ds(r, RBLK)],
                        rows_v.at[0, pl.ds(0, RBLK)])
        pltpu.sync_copy(rows_v.at[0, pl.ds(0, RBLK)],
                        acc_out.at[cid, pl.ds(r, RBLK)])

    @pl.when(sid == NS - 1)
    def _():
        r = NS * RSPAN
        pltpu.sync_copy(acc_sh.at[pl.ds(r, RTAIL)],
                        rows_v.at[0, pl.ds(0, RTAIL)])
        pltpu.sync_copy(rows_v.at[0, pl.ds(0, RTAIL)],
                        acc_out.at[cid, pl.ds(r, RTAIL)])

    @pl.when(sid == 0)
    def _():
        pltpu.sync_copy(den_sh, den_v)
        pltpu.sync_copy(den_v, den_out.at[pl.ds(cid * N, N)])


@functools.partial(
    pl.kernel,
    out_type=[
        jax.ShapeDtypeStruct((NC, N, F), jnp.float32),
        jax.ShapeDtypeStruct((NC * N,), jnp.float32),
    ],
    mesh=plsc.VectorSubcoreMesh(core_axis_name="c", subcore_axis_name="s"),
    scratch_types=[
        pltpu.VMEM((N,), jnp.float32),
        pltpu.VMEM((16,), jnp.float32),
        pltpu.VMEM((2, 2 * C), jnp.int32),
        pltpu.VMEM((2, C), jnp.int32),
        pltpu.VMEM((2, C), jnp.float32),
        pltpu.VMEM((2, C), jnp.float32),
        pltpu.VMEM((2, C), jnp.float32),
        pltpu.VMEM((2, C, F), jnp.float32),
        pltpu.SemaphoreType.DMA,
        pltpu.SemaphoreType.DMA,
        pltpu.SemaphoreType.DMA,
        pltpu.SemaphoreType.DMA,
        pltpu.SemaphoreType.DMA,
        pltpu.SemaphoreType.DMA,
        pltpu.SemaphoreType.DMA,
        pltpu.SemaphoreType.DMA,
        pltpu.VMEM_SHARED((N, F), jnp.float32),
        pltpu.VMEM_SHARED((N,), jnp.float32),
    ],
)
def _edges(ei_hbm, p_hbm, s1_hbm, s2_hbm, m_hbm, acc_out, den_out,
           den_v, m_v, sd_v, sdst_v, w_v, g1_v, g2_v, rows_v,
           si0, si1, sg0, sg1, sgs0, sgs1, ss0, ss1,
           acc_sh, den_sh):
    _edges_body(ei_hbm, p_hbm, s1_hbm, s2_hbm, m_hbm,
                acc_out, den_out,
                den_v, m_v, sd_v, sdst_v, w_v, g1_v, g2_v, rows_v,
                si0, si1, sg0, sg1, sgs0, sgs1, ss0, ss1,
                acc_sh, den_sh)


def _finish_body(acc_ref, den_ref, o_ref):
    den = den_ref[0] + den_ref[1]
    acc = acc_ref[0] + acc_ref[1]
    r = acc / jnp.maximum(den, 1e-16)[:, None]
    o_ref[...] = jnp.where(r > 0, r, jnp.exp(jnp.minimum(r, 0.0)) - 1.0)


def _finish(acc, den):
    return pl.pallas_call(
        _finish_body,
        out_shape=jax.ShapeDtypeStruct((N, F), jnp.float32),
    )(acc, den)


def kernel(h, edge_index, W, a):
    a_pad = jnp.zeros((F, F), jnp.float32)
    a_pad = a_pad.at[:, 0].set(a[0, :F]).at[:, 1].set(a[0, F:])
    p, s_pad, m128 = _project(h, W, a_pad)
    s1 = s_pad[:, 0]
    s2 = s_pad[:, 1]
    m16 = m128[0, :16]
    ei_il = edge_index.reshape(2, NCHUNK, C).transpose(1, 0, 2).reshape(2 * E)
    acc, den = _edges(ei_il, p, s1, s2, m16)
    return _finish(acc, den.reshape(NC, N))


# speedup vs baseline: 39.9562x; 1.0345x over previous
import functools

import jax
import jax.numpy as jnp
from jax import lax
from jax.experimental import pallas as pl
from jax.experimental.pallas import tpu as pltpu
from jax.experimental.pallas import tpu_sc as plsc

N = 10000
E = 320000
F = 128
NC = 2
NS = 16
NW = NC * NS
C = 128
NCHUNK = E // C
RSPAN = 624
RBLK = 104
RTAIL = N - NS * RSPAN
NEG_SLOPE = 0.2


def _proj_body(h_ref, w_ref, a2_ref, p_ref, s_ref, m_ref):
    p = jnp.dot(h_ref[...], w_ref[0], preferred_element_type=jnp.float32)
    p_ref[...] = p
    s = jnp.dot(p, a2_ref[...], preferred_element_type=jnp.float32)
    s_ref[...] = s
    m = jnp.max(s[:, 0]) + jnp.max(s[:, 1])
    m = jnp.where(m >= 0, m, NEG_SLOPE * m)
    m_ref[...] = jnp.full((1, 128), m, dtype=jnp.float32)


def _project(h, w, a2):
    return pl.pallas_call(
        _proj_body,
        out_shape=[
            jax.ShapeDtypeStruct((N, F), jnp.float32),
            jax.ShapeDtypeStruct((N, F), jnp.float32),
            jax.ShapeDtypeStruct((1, 128), jnp.float32),
        ],
    )(h, w, a2)


def _edges_body(ei_hbm, p_hbm, s1_hbm, s2_hbm, m_hbm,
                acc_out, den_out,
                den_v, m_v, src_v, dst_v, sdst_v, w_v, g1_v, g2_v, rows_v,
                si0, si1, sg0, sg1, sgs0, sgs1, ss0, ss1,
                acc_sh, den_sh):
    cid = lax.axis_index("c")
    sid = lax.axis_index("s")
    wid = sid * NC + cid
    si = (si0, si1)
    sg = (sg0, sg1)
    sgs = (sgs0, sgs1)
    ss = (ss0, ss1)

    pltpu.sync_copy(m_hbm, m_v)

    zero16 = jnp.zeros((16,), jnp.float32)

    def _zrow(i, _):
        for q in range(F // 16):
            rows_v[0, i, pl.ds(q * 16, 16)] = zero16
        return 0
    lax.fori_loop(0, RBLK, _zrow, 0)

    row0 = sid * RSPAN
    for blk in range(RSPAN // RBLK):
        pltpu.sync_copy(rows_v.at[0, pl.ds(0, RBLK)],
                        acc_sh.at[pl.ds(row0 + blk * RBLK, RBLK)])

    @pl.when(sid == NS - 1)
    def _():
        pltpu.sync_copy(rows_v.at[0, pl.ds(0, RTAIL)],
                        acc_sh.at[pl.ds(NS * RSPAN, RTAIL)])

    def _zden(i, _):
        den_v[pl.ds(i * 16, 16)] = zero16
        return 0
    lax.fori_loop(0, N // 16, _zden, 0)

    @pl.when(sid == 0)
    def _():
        pltpu.sync_copy(den_v, den_sh)

    plsc.subcore_barrier()

    mvec = m_v[...]
    n_my = (NCHUNK - wid + NW - 1) // NW

    def _issue_idx(b, j):
        off = (wid + j * NW) * C
        pltpu.async_copy(ei_hbm.at[pl.ds(off, C)], src_v.at[b], si[b])
        pltpu.async_copy(ei_hbm.at[pl.ds(E + off, C)], dst_v.at[b], si[b])

    def _wait_idx(b):
        pltpu.make_async_copy(ei_hbm.at[pl.ds(0, C)], src_v.at[b],
                              si[b]).wait()
        pltpu.make_async_copy(ei_hbm.at[pl.ds(0, C)], dst_v.at[b],
                              si[b]).wait()

    def _wait_scatter(b):
        pltpu.make_async_copy(rows_v.at[b], acc_sh.at[sdst_v.at[b]],
                              ss[b]).wait()
        pltpu.make_async_copy(w_v.at[b], den_sh.at[sdst_v.at[b]],
                              ss[b]).wait()

    def _issue_gather(b):
        pltpu.async_copy(p_hbm.at[src_v.at[b]], rows_v.at[b], sg[b])
        pltpu.async_copy(s1_hbm.at[src_v.at[b]], g1_v.at[b], sgs[b])
        pltpu.async_copy(s2_hbm.at[dst_v.at[b]], g2_v.at[b], sgs[b])

    def _wait_gather(b):
        pltpu.make_async_copy(p_hbm.at[src_v.at[b]], rows_v.at[b],
                              sg[b]).wait()
        pltpu.make_async_copy(s1_hbm.at[src_v.at[b]], g1_v.at[b],
                              sgs[b]).wait()
        pltpu.make_async_copy(s2_hbm.at[dst_v.at[b]], g2_v.at[b],
                              sgs[b]).wait()

    def _do_chunk(j, b):
        bn = 1 - b
        _wait_gather(b)

        def _wgrp(g, _):
            t = g1_v[b, pl.ds(g * 16, 16)] + g2_v[b, pl.ds(g * 16, 16)]
            t = jnp.where(t >= 0, t, NEG_SLOPE * t)
            w_v[b, pl.ds(g * 16, 16)] = jnp.exp(t - mvec)
            sdst_v[b, pl.ds(g * 16, 16)] = dst_v[b, pl.ds(g * 16, 16)]
            return 0
        lax.fori_loop(0, C // 16, _wgrp, 0)

        @pl.when(jnp.logical_and(j >= 1, j + 1 < n_my))
        def _():
            _wait_scatter(bn)

        @pl.when(j + 1 < n_my)
        def _():
            _wait_idx(bn)
            _issue_gather(bn)

        @pl.when(j + 2 < n_my)
        def _():
            _issue_idx(b, j + 2)

        def _scale(g, _):
            w16 = w_v[b, pl.ds(g * 16, 16)]
            for l in range(16):
                ws = w16[l]
                e = g * 16 + l
                for q in range(F // 16):
                    rows_v[b, e, pl.ds(q * 16, 16)] = (
                        rows_v[b, e, pl.ds(q * 16, 16)] * ws)
            return 0
        lax.fori_loop(0, C // 16, _scale, 0)

        pltpu.async_copy(rows_v.at[b], acc_sh.at[sdst_v.at[b]], ss[b],
                         add=True)
        pltpu.async_copy(w_v.at[b], den_sh.at[sdst_v.at[b]], ss[b],
                         add=True)

    _issue_idx(0, 0)
    _wait_idx(0)
    _issue_gather(0)
    @pl.when(n_my >= 2)
    def _():
        _issue_idx(1, 1)

    def _pair(p2, _):
        _do_chunk(2 * p2, 0)
        _do_chunk(2 * p2 + 1, 1)
        return 0
    lax.fori_loop(0, NCHUNK // NW // 2, _pair, 0)

    @pl.when(n_my % 2 == 1)
    def _():
        _do_chunk(n_my - 1, 0)

    _wait_scatter(0)
    _wait_scatter(1)

    plsc.subcore_barrier()

    for blk in range(RSPAN // RBLK):
        r = row0 + blk * RBLK
        pltpu.sync_copy(acc_sh.at[pl.ds(r, RBLK)],
                        rows_v.at[0, pl.ds(0, RBLK)])
        pltpu.sync_copy(rows_v.at[0, pl.ds(0, RBLK)],
                        acc_out.at[cid, pl.ds(r, RBLK)])

    @pl.when(sid == NS - 1)
    def _():
        r = NS * RSPAN
        pltpu.sync_copy(acc_sh.at[pl.ds(r, RTAIL)],
                        rows_v.at[0, pl.ds(0, RTAIL)])
        pltpu.sync_copy(rows_v.at[0, pl.ds(0, RTAIL)],
                        acc_out.at[cid, pl.ds(r, RTAIL)])

    @pl.when(sid == 0)
    def _():
        pltpu.sync_copy(den_sh, den_v)
        pltpu.sync_copy(den_v, den_out.at[pl.ds(cid * N, N)])


@functools.partial(
    pl.kernel,
    out_type=[
        jax.ShapeDtypeStruct((NC, N, F), jnp.float32),
        jax.ShapeDtypeStruct((NC * N,), jnp.float32),
    ],
    mesh=plsc.VectorSubcoreMesh(core_axis_name="c", subcore_axis_name="s"),
    scratch_types=[
        pltpu.VMEM((N,), jnp.float32),
        pltpu.VMEM((16,), jnp.float32),
        pltpu.VMEM((2, C), jnp.int32),
        pltpu.VMEM((2, C), jnp.int32),
        pltpu.VMEM((2, C), jnp.int32),
        pltpu.VMEM((2, C), jnp.float32),
        pltpu.VMEM((2, C), jnp.float32),
        pltpu.VMEM((2, C), jnp.float32),
        pltpu.VMEM((2, C, F), jnp.float32),
        pltpu.SemaphoreType.DMA,
        pltpu.SemaphoreType.DMA,
        pltpu.SemaphoreType.DMA,
        pltpu.SemaphoreType.DMA,
        pltpu.SemaphoreType.DMA,
        pltpu.SemaphoreType.DMA,
        pltpu.SemaphoreType.DMA,
        pltpu.SemaphoreType.DMA,
        pltpu.VMEM_SHARED((N, F), jnp.float32),
        pltpu.VMEM_SHARED((N,), jnp.float32),
    ],
)
def _edges(ei_hbm, p_hbm, s1_hbm, s2_hbm, m_hbm, acc_out, den_out,
           den_v, m_v, src_v, dst_v, sdst_v, w_v, g1_v, g2_v, rows_v,
           si0, si1, sg0, sg1, sgs0, sgs1, ss0, ss1,
           acc_sh, den_sh):
    _edges_body(ei_hbm, p_hbm, s1_hbm, s2_hbm, m_hbm,
                acc_out, den_out,
                den_v, m_v, src_v, dst_v, sdst_v, w_v, g1_v, g2_v, rows_v,
                si0, si1, sg0, sg1, sgs0, sgs1, ss0, ss1,
                acc_sh, den_sh)


def _finish_body(acc_ref, den_ref, o_ref):
    den = den_ref[0] + den_ref[1]
    acc = acc_ref[0] + acc_ref[1]
    r = acc / jnp.maximum(den, 1e-16)[:, None]
    o_ref[...] = jnp.where(r > 0, r, jnp.exp(jnp.minimum(r, 0.0)) - 1.0)


def _finish(acc, den):
    return pl.pallas_call(
        _finish_body,
        out_shape=jax.ShapeDtypeStruct((N, F), jnp.float32),
    )(acc, den)


def kernel(h, edge_index, W, a):
    a_pad = jnp.zeros((F, F), jnp.float32)
    a_pad = a_pad.at[:, 0].set(a[0, :F]).at[:, 1].set(a[0, F:])
    p, s_pad, m128 = _project(h, W, a_pad)
    s1 = s_pad[:, 0]
    s2 = s_pad[:, 1]
    m16 = m128[0, :16]
    acc, den = _edges(edge_index.reshape(2 * E), p, s1, s2, m16)
    return _finish(acc, den.reshape(NC, N))


# speedup vs baseline: 41.0114x; 1.0264x over previous
import functools

import jax
import jax.numpy as jnp
from jax import lax
from jax.experimental import pallas as pl
from jax.experimental.pallas import tpu as pltpu
from jax.experimental.pallas import tpu_sc as plsc

N = 10000
E = 320000
F = 128
NC = 2
NS = 16
NW = NC * NS
C = 128
NCHUNK = E // C
RSPAN = 624
RBLK = 104
RTAIL = N - NS * RSPAN
NEG_SLOPE = 0.2


def _proj_body(h_ref, w_ref, a2_ref, p_ref, s_ref, m_ref):
    p = jnp.dot(h_ref[...], w_ref[0], preferred_element_type=jnp.float32)
    p_ref[...] = p
    s = jnp.dot(p, a2_ref[...], preferred_element_type=jnp.float32)
    s_ref[...] = s
    m = jnp.max(s[:, 0]) + jnp.max(s[:, 1])
    m = jnp.where(m >= 0, m, NEG_SLOPE * m)
    m_ref[...] = jnp.full((1, 128), m, dtype=jnp.float32)


def _project(h, w, a2):
    return pl.pallas_call(
        _proj_body,
        out_shape=[
            jax.ShapeDtypeStruct((N, F), jnp.float32),
            jax.ShapeDtypeStruct((N, F), jnp.float32),
            jax.ShapeDtypeStruct((1, 128), jnp.float32),
        ],
    )(h, w, a2)


def _edges_body(ei_hbm, p_hbm, s1_hbm, s2_hbm, m_hbm,
                acc_out, den_out,
                den_v, m_v, src_v, dst_v, sdst_v, w_v, g1_v, g2_v, rows_v,
                si0, si1, sg0, sg1, sgs0, sgs1, ss0, ss1,
                acc_sh, den_sh):
    cid = lax.axis_index("c")
    sid = lax.axis_index("s")
    wid = sid * NC + cid
    si = (si0, si1)
    sg = (sg0, sg1)
    sgs = (sgs0, sgs1)
    ss = (ss0, ss1)

    pltpu.sync_copy(m_hbm, m_v)

    zero16 = jnp.zeros((16,), jnp.float32)

    def _zrow(i, _):
        for q in range(F // 16):
            rows_v[0, i, pl.ds(q * 16, 16)] = zero16
        return 0
    lax.fori_loop(0, RBLK, _zrow, 0)

    row0 = sid * RSPAN
    for blk in range(RSPAN // RBLK):
        pltpu.sync_copy(rows_v.at[0, pl.ds(0, RBLK)],
                        acc_sh.at[pl.ds(row0 + blk * RBLK, RBLK)])

    @pl.when(sid == NS - 1)
    def _():
        pltpu.sync_copy(rows_v.at[0, pl.ds(0, RTAIL)],
                        acc_sh.at[pl.ds(NS * RSPAN, RTAIL)])

    def _zden(i, _):
        den_v[pl.ds(i * 16, 16)] = zero16
        return 0
    lax.fori_loop(0, N // 16, _zden, 0)

    @pl.when(sid == 0)
    def _():
        pltpu.sync_copy(den_v, den_sh)

    plsc.subcore_barrier()

    mvec = m_v[...]
    n_my = (NCHUNK - wid + NW - 1) // NW

    def _issue_idx(b, j):
        off = (wid + j * NW) * C
        pltpu.async_copy(ei_hbm.at[pl.ds(off, C)], src_v.at[b], si[b])
        pltpu.async_copy(ei_hbm.at[pl.ds(E + off, C)], dst_v.at[b], si[b])

    def _wait_idx(b):
        pltpu.make_async_copy(ei_hbm.at[pl.ds(0, C)], src_v.at[b],
                              si[b]).wait()
        pltpu.make_async_copy(ei_hbm.at[pl.ds(0, C)], dst_v.at[b],
                              si[b]).wait()

    def _wait_scatter(b):
        pltpu.make_async_copy(rows_v.at[b], acc_sh.at[sdst_v.at[b]],
                              ss[b]).wait()
        pltpu.make_async_copy(w_v.at[b], den_sh.at[sdst_v.at[b]],
                              ss[b]).wait()

    def _issue_gather(b):
        pltpu.async_copy(p_hbm.at[src_v.at[b]], rows_v.at[b], sg[b])
        pltpu.async_copy(s1_hbm.at[src_v.at[b]], g1_v.at[b], sgs[b])
        pltpu.async_copy(s2_hbm.at[dst_v.at[b]], g2_v.at[b], sgs[b])

    def _wait_gather_scalars(b):
        pltpu.make_async_copy(s1_hbm.at[src_v.at[b]], g1_v.at[b],
                              sgs[b]).wait()
        pltpu.make_async_copy(s2_hbm.at[dst_v.at[b]], g2_v.at[b],
                              sgs[b]).wait()

    def _wait_gather_rows(b):
        pltpu.make_async_copy(p_hbm.at[src_v.at[b]], rows_v.at[b],
                              sg[b]).wait()

    def _do_chunk(j, b):
        bn = 1 - b
        _wait_gather_scalars(b)

        def _wgrp(g, _):
            t = g1_v[b, pl.ds(g * 16, 16)] + g2_v[b, pl.ds(g * 16, 16)]
            t = jnp.where(t >= 0, t, NEG_SLOPE * t)
            w_v[b, pl.ds(g * 16, 16)] = jnp.exp(t - mvec)
            sdst_v[b, pl.ds(g * 16, 16)] = dst_v[b, pl.ds(g * 16, 16)]
            return 0
        lax.fori_loop(0, C // 16, _wgrp, 0)

        @pl.when(jnp.logical_and(j >= 1, j + 1 < n_my))
        def _():
            _wait_scatter(bn)

        @pl.when(j + 1 < n_my)
        def _():
            _wait_idx(bn)
            _issue_gather(bn)

        _wait_gather_rows(b)

        @pl.when(j + 2 < n_my)
        def _():
            _issue_idx(b, j + 2)

        def _scale(g, _):
            w16 = w_v[b, pl.ds(g * 16, 16)]
            for l in range(16):
                ws = w16[l]
                e = g * 16 + l
                for q in range(F // 16):
                    rows_v[b, e, pl.ds(q * 16, 16)] = (
                        rows_v[b, e, pl.ds(q * 16, 16)] * ws)
            return 0
        lax.fori_loop(0, C // 16, _scale, 0)

        pltpu.async_copy(rows_v.at[b], acc_sh.at[sdst_v.at[b]], ss[b],
                         add=True)
        pltpu.async_copy(w_v.at[b], den_sh.at[sdst_v.at[b]], ss[b],
                         add=True)

    _issue_idx(0, 0)
    _wait_idx(0)
    _issue_gather(0)
    @pl.when(n_my >= 2)
    def _():
        _issue_idx(1, 1)

    def _pair(p2, _):
        _do_chunk(2 * p2, 0)
        _do_chunk(2 * p2 + 1, 1)
        return 0
    lax.fori_loop(0, NCHUNK // NW // 2, _pair, 0)

    @pl.when(n_my % 2 == 1)
    def _():
        _do_chunk(n_my - 1, 0)

    _wait_scatter(0)
    _wait_scatter(1)

    plsc.subcore_barrier()

    for blk in range(RSPAN // RBLK):
        r = row0 + blk * RBLK
        pltpu.sync_copy(acc_sh.at[pl.ds(r, RBLK)],
                        rows_v.at[0, pl.ds(0, RBLK)])
        pltpu.sync_copy(rows_v.at[0, pl.ds(0, RBLK)],
                        acc_out.at[cid, pl.ds(r, RBLK)])

    @pl.when(sid == NS - 1)
    def _():
        r = NS * RSPAN
        pltpu.sync_copy(acc_sh.at[pl.ds(r, RTAIL)],
                        rows_v.at[0, pl.ds(0, RTAIL)])
        pltpu.sync_copy(rows_v.at[0, pl.ds(0, RTAIL)],
                        acc_out.at[cid, pl.ds(r, RTAIL)])

    @pl.when(sid == 0)
    def _():
        pltpu.sync_copy(den_sh, den_v)
        pltpu.sync_copy(den_v, den_out.at[pl.ds(cid * N, N)])


@functools.partial(
    pl.kernel,
    out_type=[
        jax.ShapeDtypeStruct((NC, N, F), jnp.float32),
        jax.ShapeDtypeStruct((NC * N,), jnp.float32),
    ],
    mesh=plsc.VectorSubcoreMesh(core_axis_name="c", subcore_axis_name="s"),
    scratch_types=[
        pltpu.VMEM((N,), jnp.float32),
        pltpu.VMEM((16,), jnp.float32),
        pltpu.VMEM((2, C), jnp.int32),
        pltpu.VMEM((2, C), jnp.int32),
        pltpu.VMEM((2, C), jnp.int32),
        pltpu.VMEM((2, C), jnp.float32),
        pltpu.VMEM((2, C), jnp.float32),
        pltpu.VMEM((2, C), jnp.float32),
        pltpu.VMEM((2, C, F), jnp.float32),
        pltpu.SemaphoreType.DMA,
        pltpu.SemaphoreType.DMA,
        pltpu.SemaphoreType.DMA,
        pltpu.SemaphoreType.DMA,
        pltpu.SemaphoreType.DMA,
        pltpu.SemaphoreType.DMA,
        pltpu.SemaphoreType.DMA,
        pltpu.SemaphoreType.DMA,
        pltpu.VMEM_SHARED((N, F), jnp.float32),
        pltpu.VMEM_SHARED((N,), jnp.float32),
    ],
)
def _edges(ei_hbm, p_hbm, s1_hbm, s2_hbm, m_hbm, acc_out, den_out,
           den_v, m_v, src_v, dst_v, sdst_v, w_v, g1_v, g2_v, rows_v,
           si0, si1, sg0, sg1, sgs0, sgs1, ss0, ss1,
           acc_sh, den_sh):
    _edges_body(ei_hbm, p_hbm, s1_hbm, s2_hbm, m_hbm,
                acc_out, den_out,
                den_v, m_v, src_v, dst_v, sdst_v, w_v, g1_v, g2_v, rows_v,
                si0, si1, sg0, sg1, sgs0, sgs1, ss0, ss1,
                acc_sh, den_sh)


def _finish_body(acc_ref, den_ref, o_ref):
    den = den_ref[0] + den_ref[1]
    acc = acc_ref[0] + acc_ref[1]
    r = acc / jnp.maximum(den, 1e-16)[:, None]
    o_ref[...] = jnp.where(r > 0, r, jnp.exp(jnp.minimum(r, 0.0)) - 1.0)


def _finish(acc, den):
    return pl.pallas_call(
        _finish_body,
        out_shape=jax.ShapeDtypeStruct((N, F), jnp.float32),
    )(acc, den)


def kernel(h, edge_index, W, a):
    a_pad = jnp.zeros((F, F), jnp.float32)
    a_pad = a_pad.at[:, 0].set(a[0, :F]).at[:, 1].set(a[0, F:])
    p, s_pad, m128 = _project(h, W, a_pad)
    s1 = s_pad[:, 0]
    s2 = s_pad[:, 1]
    m16 = m128[0, :16]
    acc, den = _edges(edge_index.reshape(2 * E), p, s1, s2, m16)
    return _finish(acc, den.reshape(NC, N))
